# baseline (device time: 136325 ns/iter reference)
import jax
import jax.numpy as jnp
from jax import lax
from jax.experimental import pallas as pl
from jax.experimental.pallas import tpu as pltpu

N_DEV = 16
HR = N_DEV // 2
HL = N_DEV // 2 - 1


def kernel(A, B):
    m_per, k = A.shape
    _, n = B.shape

    def body(a_ref, b_ref, out_ref,
             commR, commL, sendR, recvR, sendL, recvL, ackR, ackL):
        my = lax.axis_index("i")
        left = lax.rem(my + N_DEV - 1, N_DEV)
        right = lax.rem(my + 1, N_DEV)

        barrier_sem = pltpu.get_barrier_semaphore()
        for nbr in (left, right):
            pl.semaphore_signal(
                barrier_sem, inc=1,
                device_id=(nbr,), device_id_type=pl.DeviceIdType.MESH,
            )
        pl.semaphore_wait(barrier_sem, 2)

        a_bf16 = a_ref[...].astype(jnp.bfloat16)
        commR[0] = a_bf16
        commL[0] = a_bf16
        b_bf16 = b_ref[...].astype(jnp.bfloat16)

        def dot_store(origin, chunk):
            out_ref[pl.ds(origin * m_per, m_per), :] = jnp.dot(
                chunk, b_bf16, preferred_element_type=jnp.float32
            )

        for h in range(HR):
            s, r = h % 2, (h + 1) % 2

            if h >= 1:
                pl.semaphore_wait(ackR, 1)
            rdmaR = pltpu.make_async_remote_copy(
                src_ref=commR.at[s], dst_ref=commR.at[r],
                send_sem=sendR.at[s], recv_sem=recvR.at[r],
                device_id=(right,), device_id_type=pl.DeviceIdType.MESH,
            )
            rdmaR.start()
            rdmaL = None
            if h < HL:
                if h >= 1:
                    pl.semaphore_wait(ackL, 1)
                rdmaL = pltpu.make_async_remote_copy(
                    src_ref=commL.at[s], dst_ref=commL.at[r],
                    send_sem=sendL.at[s], recv_sem=recvL.at[r],
                    device_id=(left,), device_id_type=pl.DeviceIdType.MESH,
                )
                rdmaL.start()

            if h == 0:
                dot_store(my, a_bf16)
            else:
                dot_store(lax.rem(my + N_DEV - h, N_DEV), commR[s])
                dot_store(lax.rem(my + h, N_DEV), commL[s])

            rdmaR.wait_send()
            rdmaR.wait_recv()
            if rdmaL is not None:
                rdmaL.wait_send()
                rdmaL.wait_recv()

            if h <= HR - 2:
                pl.semaphore_signal(
                    ackR, inc=1,
                    device_id=(left,), device_id_type=pl.DeviceIdType.MESH,
                )
            if h <= HL - 2:
                pl.semaphore_signal(
                    ackL, inc=1,
                    device_id=(right,), device_id_type=pl.DeviceIdType.MESH,
                )

        dot_store(lax.rem(my + N_DEV - HR, N_DEV), commR[HR % 2])

    return pl.pallas_call(
        body,
        out_shape=jax.ShapeDtypeStruct((N_DEV * m_per, n), jnp.float32),
        in_specs=[
            pl.BlockSpec(memory_space=pltpu.VMEM),
            pl.BlockSpec(memory_space=pltpu.VMEM),
        ],
        out_specs=pl.BlockSpec(memory_space=pltpu.VMEM),
        scratch_shapes=[
            pltpu.VMEM((2, m_per, k), jnp.bfloat16),
            pltpu.VMEM((2, m_per, k), jnp.bfloat16),
            pltpu.SemaphoreType.DMA((2,)),
            pltpu.SemaphoreType.DMA((2,)),
            pltpu.SemaphoreType.DMA((2,)),
            pltpu.SemaphoreType.DMA((2,)),
            pltpu.SemaphoreType.REGULAR,
            pltpu.SemaphoreType.REGULAR,
        ],
        compiler_params=pltpu.CompilerParams(
            collective_id=0, vmem_limit_bytes=100 * 1024 * 1024
        ),
    )(A, B)


# device time: 122183 ns/iter; 1.1157x vs baseline; 1.1157x over previous
import jax
import jax.numpy as jnp
from jax import lax
from jax.experimental import pallas as pl
from jax.experimental.pallas import tpu as pltpu

N_DEV = 16
HR = N_DEV // 2
HL = N_DEV // 2 - 1
NSLOT = 4


def kernel(A, B):
    m_per, k = A.shape
    _, n = B.shape

    def body(a_ref, b_ref, out_ref,
             commR, commL, sendR, recvR, sendL, recvL, ackR, ackL):
        my = lax.axis_index("i")
        left = lax.rem(my + N_DEV - 1, N_DEV)
        right = lax.rem(my + 1, N_DEV)

        barrier_sem = pltpu.get_barrier_semaphore()
        for nbr in (left, right):
            pl.semaphore_signal(
                barrier_sem, inc=1,
                device_id=(nbr,), device_id_type=pl.DeviceIdType.MESH,
            )
        pl.semaphore_wait(barrier_sem, 2)

        a_bf16 = a_ref[...].astype(jnp.bfloat16)
        commR[NSLOT - 1] = a_bf16
        commL[NSLOT - 1] = a_bf16
        b_bf16 = b_ref[...].astype(jnp.bfloat16)

        def dot_store(origin, chunk):
            out_ref[pl.ds(origin * m_per, m_per), :] = jnp.dot(
                chunk, b_bf16, preferred_element_type=jnp.float32
            )

        for h in range(HR):
            s = (h + NSLOT - 1) % NSLOT
            r = h % NSLOT

            if h >= NSLOT - 1:
                pl.semaphore_wait(ackR, 1)
            rdmaR = pltpu.make_async_remote_copy(
                src_ref=commR.at[s], dst_ref=commR.at[r],
                send_sem=sendR.at[s], recv_sem=recvR.at[r],
                device_id=(right,), device_id_type=pl.DeviceIdType.MESH,
            )
            rdmaR.start()
            rdmaL = None
            if h < HL:
                if h >= NSLOT - 1:
                    pl.semaphore_wait(ackL, 1)
                rdmaL = pltpu.make_async_remote_copy(
                    src_ref=commL.at[s], dst_ref=commL.at[r],
                    send_sem=sendL.at[s], recv_sem=recvL.at[r],
                    device_id=(left,), device_id_type=pl.DeviceIdType.MESH,
                )
                rdmaL.start()

            if h == 0:
                dot_store(my, a_bf16)
            else:
                dot_store(lax.rem(my + N_DEV - h, N_DEV), commR[s])
                dot_store(lax.rem(my + h, N_DEV), commL[s])

            rdmaR.wait_send()
            rdmaR.wait_recv()
            if rdmaL is not None:
                rdmaL.wait_send()
                rdmaL.wait_recv()

            if h <= HR - NSLOT:
                pl.semaphore_signal(
                    ackR, inc=1,
                    device_id=(left,), device_id_type=pl.DeviceIdType.MESH,
                )
            if h <= HL - NSLOT:
                pl.semaphore_signal(
                    ackL, inc=1,
                    device_id=(right,), device_id_type=pl.DeviceIdType.MESH,
                )

        dot_store(lax.rem(my + N_DEV - HR, N_DEV), commR[(HR - 1) % NSLOT])

    return pl.pallas_call(
        body,
        out_shape=jax.ShapeDtypeStruct((N_DEV * m_per, n), jnp.float32),
        in_specs=[
            pl.BlockSpec(memory_space=pltpu.VMEM),
            pl.BlockSpec(memory_space=pltpu.VMEM),
        ],
        out_specs=pl.BlockSpec(memory_space=pltpu.VMEM),
        scratch_shapes=[
            pltpu.VMEM((NSLOT, m_per, k), jnp.bfloat16),
            pltpu.VMEM((NSLOT, m_per, k), jnp.bfloat16),
            pltpu.SemaphoreType.DMA((NSLOT,)),
            pltpu.SemaphoreType.DMA((NSLOT,)),
            pltpu.SemaphoreType.DMA((NSLOT,)),
            pltpu.SemaphoreType.DMA((NSLOT,)),
            pltpu.SemaphoreType.REGULAR,
            pltpu.SemaphoreType.REGULAR,
        ],
        compiler_params=pltpu.CompilerParams(
            collective_id=0, vmem_limit_bytes=100 * 1024 * 1024
        ),
    )(A, B)
